# baseline (speedup 1.0000x reference)
import functools

import jax
import jax.numpy as jnp
import numpy as np
from jax import lax
from jax.experimental import pallas as pl
from jax.experimental.pallas import tpu as pltpu
from jax.experimental.pallas import tpu_sc as plsc

_B, _C, _T = 4, 1024, 4096

_masked_channels = np.array([
    31, 35, 45, 85, 99, 112, 121, 130, 139, 144, 148, 152, 176, 179, 188,
    189, 197, 257, 263, 268, 304, 309, 312, 315, 318, 325, 356, 366, 398,
    409, 410, 429, 446, 448, 462, 480, 487, 493, 495, 499, 501, 507, 516,
    517, 518, 520, 532, 538, 541, 543, 544, 552, 557, 567, 569, 575, 577,
    582, 591, 602, 605, 617, 649, 659, 707, 709, 712, 739, 748, 750, 753,
    762, 768, 780, 787, 790, 793, 799, 842, 846, 848, 854, 857, 864, 879,
    883, 893, 895, 901, 914, 934, 942, 955, 957, 973, 976, 981, 984, 999,
    1001, 1005, 1016], dtype=np.int64)

_NW = 32
_ROWS_PER_W = (_B * _C) // _NW
_CHUNK = 8
_NCHUNKS = _ROWS_PER_W // _CHUNK
_MIN_CCHUNKS = 14
_ZWAVE = 4
_MAX_WAVES = 5

_zidx_np = np.zeros((_NW, _MAX_WAVES, _ZWAVE), dtype=np.int32)
_wave_counts = []
for _w in range(_NW):
    _lo = _w * _ROWS_PER_W
    _b = _lo // _C
    _rows = [_b * _C + int(c) for c in _masked_channels
             if _lo <= _b * _C + int(c) < _lo + _ROWS_PER_W]
    _n_waves = -(-len(_rows) // _ZWAVE)
    assert 0 < _n_waves <= _MAX_WAVES
    _wave_counts.append(_n_waves)
    _padded = (_rows + [_rows[0]] * (_MAX_WAVES * _ZWAVE))[:_MAX_WAVES * _ZWAVE]
    _zidx_np[_w] = np.asarray(_padded, np.int32).reshape(_MAX_WAVES, _ZWAVE)
assert _wave_counts[:8] == _wave_counts[8:16] == _wave_counts[16:24] == _wave_counts[24:]
_PACKED_WAVES = 0
for _o in range(8):
    assert _wave_counts[_o] < 8
    _PACKED_WAVES |= _wave_counts[_o] << (3 * _o)

_cidx_np = np.zeros((_NW, _NCHUNKS, _CHUNK), dtype=np.int32)
_cchunk_counts = []
for _w in range(_NW):
    _lo = _w * _ROWS_PER_W
    _b = _lo // _C
    _m = set(int(_b * _C + c) for c in _masked_channels
             if _lo <= _b * _C + int(c) < _lo + _ROWS_PER_W)
    _rows = [r for r in range(_lo, _lo + _ROWS_PER_W) if r not in _m]
    _cc = -(-len(_rows) // _CHUNK)
    assert _MIN_CCHUNKS <= _cc <= _NCHUNKS
    _cchunk_counts.append(_cc)
    _padded = (_rows + [_rows[0]] * (_NCHUNKS * _CHUNK))[:_NCHUNKS * _CHUNK]
    _cidx_np[_w] = np.asarray(_padded, np.int32).reshape(_NCHUNKS, _CHUNK)
assert _cchunk_counts[:8] == _cchunk_counts[8:16] == _cchunk_counts[16:24] == _cchunk_counts[24:]
_PACKED_CC = 0
for _o in range(8):
    assert 0 <= _cchunk_counts[_o] - _MIN_CCHUNKS < 8
    _PACKED_CC |= (_cchunk_counts[_o] - _MIN_CCHUNKS) << (3 * _o)

_mesh = plsc.VectorSubcoreMesh(core_axis_name="c", subcore_axis_name="s")


@functools.partial(
    pl.kernel,
    mesh=_mesh,
    out_type=jax.ShapeDtypeStruct((_B * _C, _T), jnp.float32),
    scratch_types=[
        pltpu.VMEM((_CHUNK, _T), jnp.float32),
        pltpu.VMEM((_CHUNK, _T), jnp.float32),
        pltpu.VMEM((_ZWAVE, _T), jnp.float32),
        pltpu.VMEM((_NCHUNKS, _CHUNK), jnp.int32),
        pltpu.VMEM((_MAX_WAVES, _ZWAVE), jnp.int32),
        pltpu.SemaphoreType.DMA,
        pltpu.SemaphoreType.DMA,
        pltpu.SemaphoreType.DMA,
        pltpu.SemaphoreType.DMA,
    ],
)
def _sc_masked_copy(x_hbm, cidx_hbm, zidx_hbm, zeros_hbm, out_hbm,
                    buf0, buf1, zsrc, cidx_v, zidx_v,
                    sem_in0, sem_in1, sem_out0, sem_out1):
    wid = lax.axis_index("s") * 2 + lax.axis_index("c")
    octant3 = 3 * lax.rem(wid, 8)
    n_waves = lax.shift_right_logical(jnp.int32(_PACKED_WAVES), octant3) & 7
    n_chunks = _MIN_CCHUNKS + (
        lax.shift_right_logical(jnp.int32(_PACKED_CC), octant3) & 7)
    pltpu.sync_copy(cidx_hbm.at[wid], cidx_v)
    pltpu.sync_copy(zidx_hbm.at[wid], zidx_v)
    pltpu.sync_copy(zeros_hbm, zsrc)
    bufs = (buf0, buf1)
    sem_in = (sem_in0, sem_in1)
    sem_out = (sem_out0, sem_out1)
    out_h = [None, None]
    for k in range(_MIN_CCHUNKS):
        s = k % 2
        if out_h[s] is not None:
            out_h[s].wait()
        pltpu.async_copy(x_hbm.at[cidx_v.at[k]], bufs[s], sem_in[s]).wait()
        out_h[s] = pltpu.async_copy(bufs[s], out_hbm.at[cidx_v.at[k]], sem_out[s])
    out_h[0].wait()
    out_h[1].wait()
    for k in range(_MIN_CCHUNKS, _NCHUNKS):
        @pl.when(k < n_chunks)
        def _():
            pltpu.sync_copy(x_hbm.at[cidx_v.at[k]], buf0)
            pltpu.sync_copy(buf0, out_hbm.at[cidx_v.at[k]])
    for j in range(_MAX_WAVES):
        @pl.when(j < n_waves)
        def _():
            pltpu.sync_copy(zsrc, out_hbm.at[zidx_v.at[j]])


def kernel(x):
    B, C, T = x.shape
    x2 = x.reshape(B * C, T)
    cidx = jnp.asarray(_cidx_np)
    zidx = jnp.asarray(_zidx_np)
    zeros = jnp.zeros((_ZWAVE, T), jnp.float32)
    out = _sc_masked_copy(x2, cidx, zidx, zeros)
    return out.reshape(B, C, T)


# speedup vs baseline: 1.0726x; 1.0726x over previous
import functools

import jax
import jax.numpy as jnp
import numpy as np
from jax import lax
from jax.experimental import pallas as pl
from jax.experimental.pallas import tpu as pltpu
from jax.experimental.pallas import tpu_sc as plsc

_B, _C, _T = 4, 1024, 4096

_masked_channels = np.array([
    31, 35, 45, 85, 99, 112, 121, 130, 139, 144, 148, 152, 176, 179, 188,
    189, 197, 257, 263, 268, 304, 309, 312, 315, 318, 325, 356, 366, 398,
    409, 410, 429, 446, 448, 462, 480, 487, 493, 495, 499, 501, 507, 516,
    517, 518, 520, 532, 538, 541, 543, 544, 552, 557, 567, 569, 575, 577,
    582, 591, 602, 605, 617, 649, 659, 707, 709, 712, 739, 748, 750, 753,
    762, 768, 780, 787, 790, 793, 799, 842, 846, 848, 854, 857, 864, 879,
    883, 893, 895, 901, 914, 934, 942, 955, 957, 973, 976, 981, 984, 999,
    1001, 1005, 1016], dtype=np.int64)

_NW = 32
_ROWS_PER_W = (_B * _C) // _NW
_CHUNK = 8
_NCHUNKS = _ROWS_PER_W // _CHUNK
_ZWAVE = 4
_MAX_WAVES = 5

_zidx_np = np.zeros((_NW, _MAX_WAVES, _ZWAVE), dtype=np.int32)
_wave_counts = []
for _w in range(_NW):
    _lo = _w * _ROWS_PER_W
    _b = _lo // _C
    _rows = [_b * _C + int(c) for c in _masked_channels
             if _lo <= _b * _C + int(c) < _lo + _ROWS_PER_W]
    _n_waves = -(-len(_rows) // _ZWAVE)
    assert 0 < _n_waves <= _MAX_WAVES
    _wave_counts.append(_n_waves)
    _padded = (_rows + [_rows[0]] * (_MAX_WAVES * _ZWAVE))[:_MAX_WAVES * _ZWAVE]
    _zidx_np[_w] = np.asarray(_padded, np.int32).reshape(_MAX_WAVES, _ZWAVE)
assert _wave_counts[:8] == _wave_counts[8:16] == _wave_counts[16:24] == _wave_counts[24:]
_PACKED_WAVES = 0
for _o in range(8):
    assert _wave_counts[_o] < 8
    _PACKED_WAVES |= _wave_counts[_o] << (3 * _o)

_mesh = plsc.VectorSubcoreMesh(core_axis_name="c", subcore_axis_name="s")


@functools.partial(
    pl.kernel,
    mesh=_mesh,
    out_type=jax.ShapeDtypeStruct((_B * _C, _T), jnp.float32),
    scratch_types=[
        pltpu.VMEM((_CHUNK, _T), jnp.float32),
        pltpu.VMEM((_CHUNK, _T), jnp.float32),
        pltpu.VMEM((_ZWAVE, _T), jnp.float32),
        pltpu.VMEM((_MAX_WAVES, _ZWAVE), jnp.int32),
        pltpu.SemaphoreType.DMA,
        pltpu.SemaphoreType.DMA,
        pltpu.SemaphoreType.DMA,
        pltpu.SemaphoreType.DMA,
    ],
)
def _sc_masked_copy(x_hbm, zidx_hbm, zeros_hbm, out_hbm,
                    buf0, buf1, zsrc, zidx_v,
                    sem_in0, sem_in1, sem_out0, sem_out1):
    wid = lax.axis_index("s") * 2 + lax.axis_index("c")
    base = wid * _ROWS_PER_W
    n_waves = lax.shift_right_logical(
        jnp.int32(_PACKED_WAVES), 3 * lax.rem(wid, 8)) & 7
    pltpu.sync_copy(zidx_hbm.at[wid], zidx_v)
    pltpu.sync_copy(zeros_hbm, zsrc)
    bufs = (buf0, buf1)
    sem_in = (sem_in0, sem_in1)
    sem_out = (sem_out0, sem_out1)
    out_h = [None, None]
    for k in range(_NCHUNKS):
        s = k % 2
        if out_h[s] is not None:
            out_h[s].wait()
        rows = pl.ds(base + k * _CHUNK, _CHUNK)
        pltpu.async_copy(x_hbm.at[rows], bufs[s], sem_in[s]).wait()
        out_h[s] = pltpu.async_copy(bufs[s], out_hbm.at[rows], sem_out[s])
    out_h[0].wait()
    out_h[1].wait()
    for j in range(_MAX_WAVES):
        @pl.when(j < n_waves)
        def _():
            pltpu.sync_copy(zsrc, out_hbm.at[zidx_v.at[j]])


def kernel(x):
    B, C, T = x.shape
    x2 = x.reshape(B * C, T)
    zidx = jnp.asarray(_zidx_np)
    zeros = jnp.zeros((_ZWAVE, T), jnp.float32)
    out = _sc_masked_copy(x2, zidx, zeros)
    return out.reshape(B, C, T)


# speedup vs baseline: 1.0763x; 1.0035x over previous
import functools

import jax
import jax.numpy as jnp
import numpy as np
from jax import lax
from jax.experimental import pallas as pl
from jax.experimental.pallas import tpu as pltpu
from jax.experimental.pallas import tpu_sc as plsc

_B, _C, _T = 4, 1024, 4096

_masked_channels = np.array([
    31, 35, 45, 85, 99, 112, 121, 130, 139, 144, 148, 152, 176, 179, 188,
    189, 197, 257, 263, 268, 304, 309, 312, 315, 318, 325, 356, 366, 398,
    409, 410, 429, 446, 448, 462, 480, 487, 493, 495, 499, 501, 507, 516,
    517, 518, 520, 532, 538, 541, 543, 544, 552, 557, 567, 569, 575, 577,
    582, 591, 602, 605, 617, 649, 659, 707, 709, 712, 739, 748, 750, 753,
    762, 768, 780, 787, 790, 793, 799, 842, 846, 848, 854, 857, 864, 879,
    883, 893, 895, 901, 914, 934, 942, 955, 957, 973, 976, 981, 984, 999,
    1001, 1005, 1016], dtype=np.int64)

_NW = 32
_ROWS_PER_W = (_B * _C) // _NW
_CHUNK = 8
_NCHUNKS = _ROWS_PER_W // _CHUNK
_ZWAVE = 4
_MAX_WAVES = 5

_zidx_np = np.zeros((_NW, _MAX_WAVES, _ZWAVE), dtype=np.int32)
_wave_counts = []
for _w in range(_NW):
    _lo = _w * _ROWS_PER_W
    _b = _lo // _C
    _rows = [_b * _C + int(c) for c in _masked_channels
             if _lo <= _b * _C + int(c) < _lo + _ROWS_PER_W]
    _n_waves = -(-len(_rows) // _ZWAVE)
    assert 0 < _n_waves <= _MAX_WAVES
    _wave_counts.append(_n_waves)
    _padded = (_rows + [_rows[0]] * (_MAX_WAVES * _ZWAVE))[:_MAX_WAVES * _ZWAVE]
    _zidx_np[_w] = np.asarray(_padded, np.int32).reshape(_MAX_WAVES, _ZWAVE)
assert _wave_counts[:8] == _wave_counts[8:16] == _wave_counts[16:24] == _wave_counts[24:]
_PACKED_WAVES = 0
for _o in range(8):
    assert _wave_counts[_o] < 8
    _PACKED_WAVES |= _wave_counts[_o] << (3 * _o)

_mesh = plsc.VectorSubcoreMesh(core_axis_name="c", subcore_axis_name="s")


@functools.partial(
    pl.kernel,
    mesh=_mesh,
    out_type=jax.ShapeDtypeStruct((_B * _C, _T), jnp.float32),
    scratch_types=[
        pltpu.VMEM((_CHUNK, _T), jnp.float32),
        pltpu.VMEM((_CHUNK, _T), jnp.float32),
        pltpu.VMEM((_CHUNK, _T), jnp.float32),
        pltpu.VMEM((_ZWAVE, _T), jnp.float32),
        pltpu.VMEM((_MAX_WAVES, _ZWAVE), jnp.int32),
        pltpu.SemaphoreType.DMA,
        pltpu.SemaphoreType.DMA,
        pltpu.SemaphoreType.DMA,
        pltpu.SemaphoreType.DMA,
        pltpu.SemaphoreType.DMA,
        pltpu.SemaphoreType.DMA,
    ],
)
def _sc_masked_copy(x_hbm, zidx_hbm, zeros_hbm, out_hbm,
                    buf0, buf1, buf2, zsrc, zidx_v,
                    sem_in0, sem_in1, sem_in2, sem_out0, sem_out1, sem_out2):
    wid = lax.axis_index("s") * 2 + lax.axis_index("c")
    base = wid * _ROWS_PER_W
    n_waves = lax.shift_right_logical(
        jnp.int32(_PACKED_WAVES), 3 * lax.rem(wid, 8)) & 7
    pltpu.sync_copy(zidx_hbm.at[wid], zidx_v)
    pltpu.sync_copy(zeros_hbm, zsrc)
    bufs = (buf0, buf1, buf2)
    sem_in = (sem_in0, sem_in1, sem_in2)
    sem_out = (sem_out0, sem_out1, sem_out2)
    in_h = [None, None, None]
    out_h = [None, None, None]

    def _rows(k):
        return pl.ds(base + k * _CHUNK, _CHUNK)

    for k in range(2):
        s = k % 3
        in_h[s] = pltpu.async_copy(x_hbm.at[_rows(k)], bufs[s], sem_in[s])
    for k in range(_NCHUNKS):
        s = k % 3
        in_h[s].wait()
        out_h[s] = pltpu.async_copy(bufs[s], out_hbm.at[_rows(k)], sem_out[s])
        kn = k + 2
        if kn < _NCHUNKS:
            t = kn % 3
            if out_h[t] is not None:
                out_h[t].wait()
            in_h[t] = pltpu.async_copy(x_hbm.at[_rows(kn)], bufs[t], sem_in[t])
    for h in out_h:
        if h is not None:
            h.wait()
    for j in range(_MAX_WAVES):
        @pl.when(j < n_waves)
        def _():
            pltpu.sync_copy(zsrc, out_hbm.at[zidx_v.at[j]])


def kernel(x):
    B, C, T = x.shape
    x2 = x.reshape(B * C, T)
    zidx = jnp.asarray(_zidx_np)
    zeros = jnp.zeros((_ZWAVE, T), jnp.float32)
    out = _sc_masked_copy(x2, zidx, zeros)
    return out.reshape(B, C, T)


# speedup vs baseline: 1.0997x; 1.0217x over previous
import functools

import jax
import jax.numpy as jnp
import numpy as np
from jax import lax
from jax.experimental import pallas as pl
from jax.experimental.pallas import tpu as pltpu
from jax.experimental.pallas import tpu_sc as plsc

_B, _C, _T = 4, 1024, 4096

_masked_channels = np.array([
    31, 35, 45, 85, 99, 112, 121, 130, 139, 144, 148, 152, 176, 179, 188,
    189, 197, 257, 263, 268, 304, 309, 312, 315, 318, 325, 356, 366, 398,
    409, 410, 429, 446, 448, 462, 480, 487, 493, 495, 499, 501, 507, 516,
    517, 518, 520, 532, 538, 541, 543, 544, 552, 557, 567, 569, 575, 577,
    582, 591, 602, 605, 617, 649, 659, 707, 709, 712, 739, 748, 750, 753,
    762, 768, 780, 787, 790, 793, 799, 842, 846, 848, 854, 857, 864, 879,
    883, 893, 895, 901, 914, 934, 942, 955, 957, 973, 976, 981, 984, 999,
    1001, 1005, 1016], dtype=np.int64)

_NW = 32
_ROWS_PER_W = (_B * _C) // _NW
_CHUNK = 4
_NCHUNKS = _ROWS_PER_W // _CHUNK
_SLOTS = 6
_LOOKAHEAD = 4
_ZWAVE = 4
_MAX_WAVES = 5

_zidx_np = np.zeros((_NW, _MAX_WAVES, _ZWAVE), dtype=np.int32)
_wave_counts = []
for _w in range(_NW):
    _lo = _w * _ROWS_PER_W
    _b = _lo // _C
    _rows = [_b * _C + int(c) for c in _masked_channels
             if _lo <= _b * _C + int(c) < _lo + _ROWS_PER_W]
    _n_waves = -(-len(_rows) // _ZWAVE)
    assert 0 < _n_waves <= _MAX_WAVES
    _wave_counts.append(_n_waves)
    _padded = (_rows + [_rows[0]] * (_MAX_WAVES * _ZWAVE))[:_MAX_WAVES * _ZWAVE]
    _zidx_np[_w] = np.asarray(_padded, np.int32).reshape(_MAX_WAVES, _ZWAVE)
assert _wave_counts[:8] == _wave_counts[8:16] == _wave_counts[16:24] == _wave_counts[24:]
_PACKED_WAVES = 0
for _o in range(8):
    assert _wave_counts[_o] < 8
    _PACKED_WAVES |= _wave_counts[_o] << (3 * _o)

_mesh = plsc.VectorSubcoreMesh(core_axis_name="c", subcore_axis_name="s")


@functools.partial(
    pl.kernel,
    mesh=_mesh,
    out_type=jax.ShapeDtypeStruct((_B * _C, _T), jnp.float32),
    scratch_types=(
        [pltpu.VMEM((_CHUNK, _T), jnp.float32)] * _SLOTS
        + [pltpu.VMEM((_ZWAVE, _T), jnp.float32),
           pltpu.VMEM((_MAX_WAVES, _ZWAVE), jnp.int32)]
        + [pltpu.SemaphoreType.DMA] * (2 * _SLOTS)
    ),
)
def _sc_masked_copy(x_hbm, zidx_hbm, zeros_hbm, out_hbm, *scratch):
    bufs = scratch[:_SLOTS]
    zsrc = scratch[_SLOTS]
    zidx_v = scratch[_SLOTS + 1]
    sem_in = scratch[_SLOTS + 2:2 * _SLOTS + 2]
    sem_out = scratch[2 * _SLOTS + 2:]
    wid = lax.axis_index("s") * 2 + lax.axis_index("c")
    base = wid * _ROWS_PER_W
    n_waves = lax.shift_right_logical(
        jnp.int32(_PACKED_WAVES), 3 * lax.rem(wid, 8)) & 7
    pltpu.sync_copy(zidx_hbm.at[wid], zidx_v)
    pltpu.sync_copy(zeros_hbm, zsrc)
    in_h = [None] * _SLOTS
    out_h = [None] * _SLOTS

    def _rows(k):
        return pl.ds(base + k * _CHUNK, _CHUNK)

    for k in range(_LOOKAHEAD):
        s = k % _SLOTS
        in_h[s] = pltpu.async_copy(x_hbm.at[_rows(k)], bufs[s], sem_in[s])
    for k in range(_NCHUNKS):
        s = k % _SLOTS
        in_h[s].wait()
        out_h[s] = pltpu.async_copy(bufs[s], out_hbm.at[_rows(k)], sem_out[s])
        kn = k + _LOOKAHEAD
        if kn < _NCHUNKS:
            t = kn % _SLOTS
            if out_h[t] is not None:
                out_h[t].wait()
            in_h[t] = pltpu.async_copy(x_hbm.at[_rows(kn)], bufs[t], sem_in[t])
    for h in out_h:
        if h is not None:
            h.wait()
    for j in range(_MAX_WAVES):
        @pl.when(j < n_waves)
        def _():
            pltpu.sync_copy(zsrc, out_hbm.at[zidx_v.at[j]])


def kernel(x):
    B, C, T = x.shape
    x2 = x.reshape(B * C, T)
    zidx = jnp.asarray(_zidx_np)
    zeros = jnp.zeros((_ZWAVE, T), jnp.float32)
    out = _sc_masked_copy(x2, zidx, zeros)
    return out.reshape(B, C, T)
